# Initial kernel scaffold; baseline (speedup 1.0000x reference)
import jax
import jax.numpy as jnp
from jax.experimental import pallas as pl


def kernel(x_movie, x_director, x_actor, ei_movie_director, ei_director_movie, ei_movie_actor, ei_actor_movie, W1r_movie_director, b1_movie_director, W1s_movie_director, W2r_movie_director, b2_movie_director, W2s_movie_director, W1r_director_movie, b1_director_movie, W1s_director_movie, W2r_director_movie, b2_director_movie, W2s_director_movie, W1r_movie_actor, b1_movie_actor, W1s_movie_actor, W2r_movie_actor, b2_movie_actor, W2s_movie_actor, W1r_actor_movie, b1_actor_movie, W1s_actor_movie, W2r_actor_movie, b2_actor_movie, W2s_actor_movie):
    raise NotImplementedError("write your pallas kernel here")


# speedup vs baseline: 2.2671x; 2.2671x over previous
import functools

import jax
import jax.numpy as jnp
from jax import lax
from jax.experimental import pallas as pl
from jax.experimental.pallas import tpu as pltpu
from jax.experimental.pallas import tpu_sc as plsc

N_MOVIE, N_DIRECTOR, N_ACTOR = 50000, 10000, 50000
D_IN, H, OUT = 128, 64, 64
HALF = 32

NUM_TILES = 16
LANE = 128
EDGE_ALIGN = NUM_TILES * LANE

ACC_M = 50048
ACC_D = 10240

E_MD = 200704
E_DM = 200704
E_MA = 100352
E_AM = 100352


def _round_up(x, m):
    return (x + m - 1) // m * m


_BN = 1024


def _full(shape):
    return pl.BlockSpec(shape, lambda i: (0,) * len(shape))


def _rows(shape):
    return pl.BlockSpec(shape, lambda i: (i,) + (0,) * (len(shape) - 1))


def _tc_project(x, Ws):
    n = x.shape[0]
    nb = pl.cdiv(n, _BN)

    def body(*refs):
        x_ref = refs[0]
        w_refs = refs[1:1 + len(Ws)]
        o_refs = refs[1 + len(Ws):]
        xb = x_ref[...]
        for t, w_ref in enumerate(w_refs):
            z = jnp.dot(xb, w_ref[...], preferred_element_type=jnp.float32)
            o_refs[2 * t][...] = z[:, :HALF]
            o_refs[2 * t + 1][...] = z[:, HALF:]

    out_shape = []
    out_specs = []
    for _ in Ws:
        out_shape += [jax.ShapeDtypeStruct((n, HALF), jnp.float32)] * 2
        out_specs += [_rows((_BN, HALF))] * 2
    outs = pl.pallas_call(
        body,
        grid=(nb,),
        in_specs=[_rows((_BN, x.shape[1]))] + [_full(w.shape) for w in Ws],
        out_specs=out_specs,
        out_shape=out_shape,
    )(x, *Ws)
    return [(outs[2 * t], outs[2 * t + 1]) for t in range(len(Ws))]


def _tc_combine(agg_lo, agg_hi, x, Ws_list, b_list, W2_list, relu):
    n = x.shape[0]
    nb = pl.cdiv(n, _BN)
    nws, nb_, nw2 = len(Ws_list), len(b_list), len(W2_list)

    def body(*refs):
        agg_lo_ref, agg_hi_ref, x_ref = refs[:3]
        ws_refs = refs[3:3 + nws]
        b_refs = refs[3 + nws:3 + nws + nb_]
        w2_refs = refs[3 + nws + nb_:3 + nws + nb_ + nw2]
        y_ref = refs[3 + nws + nb_ + nw2]
        z_refs = refs[3 + nws + nb_ + nw2 + 1:]
        ws = ws_refs[0][...]
        for r in ws_refs[1:]:
            ws = ws + r[...]
        bb = b_refs[0][...]
        for r in b_refs[1:]:
            bb = bb + r[...]
        agg = jnp.concatenate([agg_lo_ref[...], agg_hi_ref[...]], axis=1)
        y = agg + bb + jnp.dot(x_ref[...], ws, preferred_element_type=jnp.float32)
        if relu:
            y = jnp.maximum(y, 0.0)
        y_ref[...] = y
        for t, w2_ref in enumerate(w2_refs):
            z = jnp.dot(y, w2_ref[...], preferred_element_type=jnp.float32)
            z_refs[2 * t][...] = z[:, :HALF]
            z_refs[2 * t + 1][...] = z[:, HALF:]

    in_specs = ([_rows((_BN, HALF))] * 2 + [_rows((_BN, x.shape[1]))]
                + [_full(w.shape) for w in Ws_list]
                + [_full((1, H))] * nb_
                + [_full(w.shape) for w in W2_list])
    out_shape = [jax.ShapeDtypeStruct((n, H), jnp.float32)]
    out_specs = [_rows((_BN, H))]
    for _ in W2_list:
        out_shape += [jax.ShapeDtypeStruct((n, HALF), jnp.float32)] * 2
        out_specs += [_rows((_BN, HALF))] * 2
    outs = pl.pallas_call(
        body,
        grid=(nb,),
        in_specs=in_specs,
        out_specs=out_specs,
        out_shape=out_shape,
    )(agg_lo, agg_hi, x, *Ws_list, *[b.reshape(1, H) for b in b_list], *W2_list)
    y = outs[0]
    zs = [(outs[1 + 2 * t], outs[2 + 2 * t]) for t in range(nw2)]
    return y, zs


def _sc_aggregate(edges, z_md, z_dm, z_ma, z_am):
    mesh = plsc.VectorSubcoreMesh(core_axis_name="c", subcore_axis_name="s")

    rpt_md = E_MD // EDGE_ALIGN
    rpt_dm = E_DM // EDGE_ALIGN
    rpt_ma = E_MA // EDGE_ALIGN
    rpt_am = E_AM // EDGE_ALIGN

    def body(src_md, dst_md, src_dm, dst_dm, src_ma, dst_ma, src_am, dst_am,
             zmd_lo, zmd_hi, zdm_lo, zdm_hi, zma_lo, zma_hi, zam_lo, zam_hi,
             out_m_lo, out_m_hi, out_d_lo, out_d_hi, out_a_lo, out_a_hi,
             accA, accB, sidx, didx, rowbuf, sem):
        cid = lax.axis_index("c")
        sid = lax.axis_index("s")
        zero16 = jnp.zeros((16,), jnp.float32)

        def zero_rowbuf():
            def zrow(i, carry):
                rowbuf[i, pl.ds(0, 16)] = zero16
                rowbuf[i, pl.ds(16, 16)] = zero16
                return carry

            lax.fori_loop(0, LANE, zrow, 0)

        def zero_acc(acc_r, rows_per_tile):
            base = sid * rows_per_tile
            nfull, rem = divmod(rows_per_tile, LANE)
            for k in range(nfull):
                pltpu.sync_copy(rowbuf, acc_r.at[pl.ds(base + k * LANE, LANE)])
            if rem:
                pltpu.sync_copy(rowbuf.at[pl.ds(0, rem)],
                                acc_r.at[pl.ds(base + nfull * LANE, rem)])

        def run_pass(src_r, dst_r, z_r, acc_r, rows_per_tile):
            base = sid * rows_per_tile

            def step(i, carry):
                pltpu.sync_copy(src_r.at[base + i], sidx)
                pltpu.sync_copy(dst_r.at[base + i], didx)
                pltpu.async_copy(z_r.at[sidx], rowbuf, sem).wait()
                pltpu.sync_copy(rowbuf, acc_r.at[didx], add=True)
                return carry

            lax.fori_loop(0, rows_per_tile, step, 0)

        def drain(acc_r, out_r, rows_per_tile):
            base = sid * rows_per_tile
            pltpu.sync_copy(acc_r.at[pl.ds(base, rows_per_tile)],
                            out_r.at[pl.ds(base, rows_per_tile)])

        def run_all(zdm, zam, zmd, zma, out_m, out_d, out_a):
            zero_rowbuf()
            zero_acc(accA, ACC_M // NUM_TILES)
            zero_acc(accB, ACC_D // NUM_TILES)
            plsc.subcore_barrier()
            run_pass(src_dm, dst_dm, zdm, accA, rpt_dm)
            run_pass(src_am, dst_am, zam, accA, rpt_am)
            run_pass(src_md, dst_md, zmd, accB, rpt_md)
            plsc.subcore_barrier()
            drain(accA, out_m, ACC_M // NUM_TILES)
            drain(accB, out_d, ACC_D // NUM_TILES)
            plsc.subcore_barrier()
            zero_rowbuf()
            zero_acc(accA, ACC_M // NUM_TILES)
            plsc.subcore_barrier()
            run_pass(src_ma, dst_ma, zma, accA, rpt_ma)
            plsc.subcore_barrier()
            drain(accA, out_a, ACC_M // NUM_TILES)

        @pl.when(cid == 0)
        def _():
            run_all(zdm_lo, zam_lo, zmd_lo, zma_lo, out_m_lo, out_d_lo, out_a_lo)

        @pl.when(cid == 1)
        def _():
            run_all(zdm_hi, zam_hi, zmd_hi, zma_hi, out_m_hi, out_d_hi, out_a_hi)

    out_type = [
        jax.ShapeDtypeStruct((ACC_M, HALF), jnp.float32),
        jax.ShapeDtypeStruct((ACC_M, HALF), jnp.float32),
        jax.ShapeDtypeStruct((ACC_D, HALF), jnp.float32),
        jax.ShapeDtypeStruct((ACC_D, HALF), jnp.float32),
        jax.ShapeDtypeStruct((ACC_M, HALF), jnp.float32),
        jax.ShapeDtypeStruct((ACC_M, HALF), jnp.float32),
    ]
    scratch_types = [
        pltpu.VMEM_SHARED((ACC_M, HALF), jnp.float32),
        pltpu.VMEM_SHARED((ACC_D, HALF), jnp.float32),
        pltpu.VMEM((LANE,), jnp.int32),
        pltpu.VMEM((LANE,), jnp.int32),
        pltpu.VMEM((LANE, HALF), jnp.float32),
        pltpu.SemaphoreType.DMA,
    ]
    fn = pl.kernel(body, out_type=out_type, mesh=mesh,
                   scratch_types=scratch_types,
                   compiler_params=pltpu.CompilerParams(
                       use_tc_tiling_on_sc=False))
    outs = fn(*edges[("movie", "director")], *edges[("director", "movie")],
              *edges[("movie", "actor")], *edges[("actor", "movie")],
              *z_md, *z_dm, *z_ma, *z_am)
    return (outs[0], outs[1]), (outs[2], outs[3]), (outs[4], outs[5])


def _prep_edges(ei, e_pad, dummy):
    e = ei.shape[1]
    src = jnp.pad(ei[0], (0, e_pad - e))
    dst = jnp.pad(ei[1], (0, e_pad - e), constant_values=dummy)
    return src.reshape(-1, LANE), dst.reshape(-1, LANE)


def kernel(x_movie, x_director, x_actor, ei_movie_director, ei_director_movie, ei_movie_actor, ei_actor_movie, W1r_movie_director, b1_movie_director, W1s_movie_director, W2r_movie_director, b2_movie_director, W2s_movie_director, W1r_director_movie, b1_director_movie, W1s_director_movie, W2r_director_movie, b2_director_movie, W2s_director_movie, W1r_movie_actor, b1_movie_actor, W1s_movie_actor, W2r_movie_actor, b2_movie_actor, W2s_movie_actor, W1r_actor_movie, b1_actor_movie, W1s_actor_movie, W2r_actor_movie, b2_actor_movie, W2s_actor_movie):
    edges = {
        ("movie", "director"): _prep_edges(ei_movie_director, E_MD, N_DIRECTOR),
        ("director", "movie"): _prep_edges(ei_director_movie, E_DM, N_MOVIE),
        ("movie", "actor"): _prep_edges(ei_movie_actor, E_MA, N_ACTOR),
        ("actor", "movie"): _prep_edges(ei_actor_movie, E_AM, N_MOVIE),
    }

    z1_md, z1_ma = _tc_project(x_movie, [W1r_movie_director, W1r_movie_actor])
    (z1_dm,) = _tc_project(x_director, [W1r_director_movie])
    (z1_am,) = _tc_project(x_actor, [W1r_actor_movie])

    agg1_m, agg1_d, agg1_a = _sc_aggregate(edges, z1_md, z1_dm, z1_ma, z1_am)

    h_m, (z2_md, z2_ma) = _tc_combine(
        agg1_m[0], agg1_m[1], x_movie,
        [W1s_director_movie, W1s_actor_movie],
        [b1_director_movie, b1_actor_movie],
        [W2r_movie_director, W2r_movie_actor], relu=True)
    h_d, (z2_dm,) = _tc_combine(
        agg1_d[0], agg1_d[1], x_director,
        [W1s_movie_director], [b1_movie_director],
        [W2r_director_movie], relu=True)
    h_a, (z2_am,) = _tc_combine(
        agg1_a[0], agg1_a[1], x_actor,
        [W1s_movie_actor], [b1_movie_actor],
        [W2r_actor_movie], relu=True)

    agg2_m, agg2_d, agg2_a = _sc_aggregate(edges, z2_md, z2_dm, z2_ma, z2_am)

    o_m, _ = _tc_combine(
        agg2_m[0], agg2_m[1], h_m,
        [W2s_director_movie, W2s_actor_movie],
        [b2_director_movie, b2_actor_movie], [], relu=False)
    o_d, _ = _tc_combine(
        agg2_d[0], agg2_d[1], h_d,
        [W2s_movie_director], [b2_movie_director], [], relu=False)
    o_a, _ = _tc_combine(
        agg2_a[0], agg2_a[1], h_a,
        [W2s_movie_actor], [b2_movie_actor], [], relu=False)

    return o_m, o_d, o_a


# speedup vs baseline: 3.3661x; 1.4848x over previous
import functools

import jax
import jax.numpy as jnp
from jax import lax
from jax.experimental import pallas as pl
from jax.experimental.pallas import tpu as pltpu
from jax.experimental.pallas import tpu_sc as plsc

N_MOVIE, N_DIRECTOR, N_ACTOR = 50000, 10000, 50000
D_IN, H, OUT = 128, 64, 64
HALF = 32

NUM_TILES = 16
LANE = 128
EDGE_ALIGN = NUM_TILES * LANE

ACC_M = 50048
ACC_D = 10240

E_MD = 200704
E_DM = 200704
E_MA = 102400
E_AM = 102400


def _round_up(x, m):
    return (x + m - 1) // m * m


_BN = 1024


def _full(shape):
    return pl.BlockSpec(shape, lambda i: (0,) * len(shape))


def _rows(shape):
    return pl.BlockSpec(shape, lambda i: (i,) + (0,) * (len(shape) - 1))


def _tc_project(x, Ws):
    n = x.shape[0]
    nb = pl.cdiv(n, _BN)

    def body(*refs):
        x_ref = refs[0]
        w_refs = refs[1:1 + len(Ws)]
        o_refs = refs[1 + len(Ws):]
        xb = x_ref[...]
        for t, w_ref in enumerate(w_refs):
            z = jnp.dot(xb, w_ref[...], preferred_element_type=jnp.float32)
            o_refs[2 * t][...] = z[:, :HALF]
            o_refs[2 * t + 1][...] = z[:, HALF:]

    out_shape = []
    out_specs = []
    for _ in Ws:
        out_shape += [jax.ShapeDtypeStruct((n, HALF), jnp.float32)] * 2
        out_specs += [_rows((_BN, HALF))] * 2
    outs = pl.pallas_call(
        body,
        grid=(nb,),
        in_specs=[_rows((_BN, x.shape[1]))] + [_full(w.shape) for w in Ws],
        out_specs=out_specs,
        out_shape=out_shape,
    )(x, *Ws)
    return [(outs[2 * t], outs[2 * t + 1]) for t in range(len(Ws))]


def _tc_combine(agg_lo, agg_hi, x, Ws_list, b_list, W2_list, relu):
    n = x.shape[0]
    nb = pl.cdiv(n, _BN)
    nws, nb_, nw2 = len(Ws_list), len(b_list), len(W2_list)

    def body(*refs):
        agg_lo_ref, agg_hi_ref, x_ref = refs[:3]
        ws_refs = refs[3:3 + nws]
        b_refs = refs[3 + nws:3 + nws + nb_]
        w2_refs = refs[3 + nws + nb_:3 + nws + nb_ + nw2]
        y_ref = refs[3 + nws + nb_ + nw2]
        z_refs = refs[3 + nws + nb_ + nw2 + 1:]
        ws = ws_refs[0][...]
        for r in ws_refs[1:]:
            ws = ws + r[...]
        bb = b_refs[0][...]
        for r in b_refs[1:]:
            bb = bb + r[...]
        agg = jnp.concatenate([agg_lo_ref[...], agg_hi_ref[...]], axis=1)
        y = agg + bb + jnp.dot(x_ref[...], ws, preferred_element_type=jnp.float32)
        if relu:
            y = jnp.maximum(y, 0.0)
        y_ref[...] = y
        for t, w2_ref in enumerate(w2_refs):
            z = jnp.dot(y, w2_ref[...], preferred_element_type=jnp.float32)
            z_refs[2 * t][...] = z[:, :HALF]
            z_refs[2 * t + 1][...] = z[:, HALF:]

    in_specs = ([_rows((_BN, HALF))] * 2 + [_rows((_BN, x.shape[1]))]
                + [_full(w.shape) for w in Ws_list]
                + [_full((1, H))] * nb_
                + [_full(w.shape) for w in W2_list])
    out_shape = [jax.ShapeDtypeStruct((n, H), jnp.float32)]
    out_specs = [_rows((_BN, H))]
    for _ in W2_list:
        out_shape += [jax.ShapeDtypeStruct((n, HALF), jnp.float32)] * 2
        out_specs += [_rows((_BN, HALF))] * 2
    outs = pl.pallas_call(
        body,
        grid=(nb,),
        in_specs=in_specs,
        out_specs=out_specs,
        out_shape=out_shape,
    )(agg_lo, agg_hi, x, *Ws_list, *[b.reshape(1, H) for b in b_list], *W2_list)
    y = outs[0]
    zs = [(outs[1 + 2 * t], outs[2 + 2 * t]) for t in range(nw2)]
    return y, zs


def _sc_aggregate(edges, z_md, z_dm, z_ma, z_am):
    mesh = plsc.VectorSubcoreMesh(core_axis_name="c", subcore_axis_name="s")

    rpt_md = E_MD // EDGE_ALIGN
    rpt_dm = E_DM // EDGE_ALIGN
    rpt_ma = E_MA // EDGE_ALIGN
    rpt_am = E_AM // EDGE_ALIGN

    def body(ei_md, ei_dm, ei_ma, ei_am,
             zmd_lo, zmd_hi, zdm_lo, zdm_hi, zma_lo, zma_hi, zam_lo, zam_hi,
             out_m_lo, out_m_hi, out_d_lo, out_d_hi, out_a_lo, out_a_hi,
             acc, zbuf, ibuf, rowbuf, isem, gsem):
        cid = lax.axis_index("c")
        sid = lax.axis_index("s")
        zero16 = jnp.zeros((16,), jnp.float32)

        def zero_zbuf():
            def zrow(i, carry):
                zbuf[i, pl.ds(0, 16)] = zero16
                zbuf[i, pl.ds(16, 16)] = zero16
                return carry

            lax.fori_loop(0, LANE, zrow, 0)

        def zero_acc(rows_per_tile):
            base = sid * rows_per_tile
            nfull, rem = divmod(rows_per_tile, LANE)
            for k in range(nfull):
                pltpu.sync_copy(zbuf, acc.at[pl.ds(base + k * LANE, LANE)])
            if rem:
                pltpu.sync_copy(zbuf.at[pl.ds(0, rem)],
                                acc.at[pl.ds(base + nfull * LANE, rem)])

        def run_pass(ei_r, z_r, n):
            base = sid * n
            pltpu.async_copy(ei_r.at[base], ibuf.at[0], isem)
            pltpu.async_copy(ei_r.at[base + 1], ibuf.at[1], isem)

            def step(i, carry):
                m = lax.rem(i, 4)
                g = lax.rem(i, 2)
                pltpu.make_async_copy(ei_r.at[base], ibuf.at[0], isem).wait()
                pltpu.async_copy(z_r.at[ibuf.at[m, 0]], rowbuf.at[g], gsem)

                @pl.when(i + 2 < n)
                def _():
                    pltpu.async_copy(ei_r.at[base + i + 2],
                                     ibuf.at[lax.rem(i + 2, 4)], isem)

                @pl.when(i >= 1)
                def _():
                    mp = lax.rem(i + 3, 4)
                    gp = lax.rem(i + 1, 2)
                    pltpu.make_async_copy(z_r.at[pl.ds(0, LANE)],
                                          rowbuf.at[0], gsem).wait()
                    pltpu.sync_copy(rowbuf.at[gp], acc.at[ibuf.at[mp, 1]],
                                    add=True)

                return carry

            lax.fori_loop(0, n, step, 0)
            pltpu.make_async_copy(z_r.at[pl.ds(0, LANE)],
                                  rowbuf.at[0], gsem).wait()
            pltpu.sync_copy(rowbuf.at[(n - 1) % 2],
                            acc.at[ibuf.at[(n - 1) % 4, 1]], add=True)

        def drain(out_r, rows_per_tile):
            base = sid * rows_per_tile
            pltpu.sync_copy(acc.at[pl.ds(base, rows_per_tile)],
                            out_r.at[pl.ds(base, rows_per_tile)])

        def run_all(zdm, zam, zmd, zma, out_m, out_d, out_a):
            zero_zbuf()
            zero_acc(ACC_M // NUM_TILES)
            plsc.subcore_barrier()
            run_pass(ei_dm, zdm, rpt_dm)
            run_pass(ei_am, zam, rpt_am)
            plsc.subcore_barrier()
            drain(out_m, ACC_M // NUM_TILES)
            plsc.subcore_barrier()
            zero_acc(ACC_D // NUM_TILES)
            plsc.subcore_barrier()
            run_pass(ei_md, zmd, rpt_md)
            plsc.subcore_barrier()
            drain(out_d, ACC_D // NUM_TILES)
            plsc.subcore_barrier()
            zero_acc(ACC_M // NUM_TILES)
            plsc.subcore_barrier()
            run_pass(ei_ma, zma, rpt_ma)
            plsc.subcore_barrier()
            drain(out_a, ACC_M // NUM_TILES)

        @pl.when(cid == 0)
        def _():
            run_all(zdm_lo, zam_lo, zmd_lo, zma_lo, out_m_lo, out_d_lo, out_a_lo)

        @pl.when(cid == 1)
        def _():
            run_all(zdm_hi, zam_hi, zmd_hi, zma_hi, out_m_hi, out_d_hi, out_a_hi)

    out_type = [
        jax.ShapeDtypeStruct((ACC_M, HALF), jnp.float32),
        jax.ShapeDtypeStruct((ACC_M, HALF), jnp.float32),
        jax.ShapeDtypeStruct((ACC_D, HALF), jnp.float32),
        jax.ShapeDtypeStruct((ACC_D, HALF), jnp.float32),
        jax.ShapeDtypeStruct((ACC_M, HALF), jnp.float32),
        jax.ShapeDtypeStruct((ACC_M, HALF), jnp.float32),
    ]
    scratch_types = [
        pltpu.VMEM_SHARED((ACC_M, HALF), jnp.float32),
        pltpu.VMEM((LANE, HALF), jnp.float32),
        pltpu.VMEM((4, 2, LANE), jnp.int32),
        pltpu.VMEM((2, LANE, HALF), jnp.float32),
        pltpu.SemaphoreType.DMA,
        pltpu.SemaphoreType.DMA,
    ]
    fn = pl.kernel(body, out_type=out_type, mesh=mesh,
                   scratch_types=scratch_types,
                   compiler_params=pltpu.CompilerParams(
                       use_tc_tiling_on_sc=False))
    outs = fn(edges[("movie", "director")], edges[("director", "movie")],
              edges[("movie", "actor")], edges[("actor", "movie")],
              *z_md, *z_dm, *z_ma, *z_am)
    return (outs[0], outs[1]), (outs[2], outs[3]), (outs[4], outs[5])


def _prep_edges(ei, e_pad, dummy):
    e = ei.shape[1]
    src = jnp.pad(ei[0], (0, e_pad - e))
    dst = jnp.pad(ei[1], (0, e_pad - e), constant_values=dummy)
    return jnp.stack([src.reshape(-1, LANE), dst.reshape(-1, LANE)], axis=1)


def kernel(x_movie, x_director, x_actor, ei_movie_director, ei_director_movie, ei_movie_actor, ei_actor_movie, W1r_movie_director, b1_movie_director, W1s_movie_director, W2r_movie_director, b2_movie_director, W2s_movie_director, W1r_director_movie, b1_director_movie, W1s_director_movie, W2r_director_movie, b2_director_movie, W2s_director_movie, W1r_movie_actor, b1_movie_actor, W1s_movie_actor, W2r_movie_actor, b2_movie_actor, W2s_movie_actor, W1r_actor_movie, b1_actor_movie, W1s_actor_movie, W2r_actor_movie, b2_actor_movie, W2s_actor_movie):
    edges = {
        ("movie", "director"): _prep_edges(ei_movie_director, E_MD, N_DIRECTOR),
        ("director", "movie"): _prep_edges(ei_director_movie, E_DM, N_MOVIE),
        ("movie", "actor"): _prep_edges(ei_movie_actor, E_MA, N_ACTOR),
        ("actor", "movie"): _prep_edges(ei_actor_movie, E_AM, N_MOVIE),
    }

    z1_md, z1_ma = _tc_project(x_movie, [W1r_movie_director, W1r_movie_actor])
    (z1_dm,) = _tc_project(x_director, [W1r_director_movie])
    (z1_am,) = _tc_project(x_actor, [W1r_actor_movie])

    agg1_m, agg1_d, agg1_a = _sc_aggregate(edges, z1_md, z1_dm, z1_ma, z1_am)

    h_m, (z2_md, z2_ma) = _tc_combine(
        agg1_m[0], agg1_m[1], x_movie,
        [W1s_director_movie, W1s_actor_movie],
        [b1_director_movie, b1_actor_movie],
        [W2r_movie_director, W2r_movie_actor], relu=True)
    h_d, (z2_dm,) = _tc_combine(
        agg1_d[0], agg1_d[1], x_director,
        [W1s_movie_director], [b1_movie_director],
        [W2r_director_movie], relu=True)
    h_a, (z2_am,) = _tc_combine(
        agg1_a[0], agg1_a[1], x_actor,
        [W1s_movie_actor], [b1_movie_actor],
        [W2r_actor_movie], relu=True)

    agg2_m, agg2_d, agg2_a = _sc_aggregate(edges, z2_md, z2_dm, z2_ma, z2_am)

    o_m, _ = _tc_combine(
        agg2_m[0], agg2_m[1], h_m,
        [W2s_director_movie, W2s_actor_movie],
        [b2_director_movie, b2_actor_movie], [], relu=False)
    o_d, _ = _tc_combine(
        agg2_d[0], agg2_d[1], h_d,
        [W2s_movie_director], [b2_movie_director], [], relu=False)
    o_a, _ = _tc_combine(
        agg2_a[0], agg2_a[1], h_a,
        [W2s_movie_actor], [b2_movie_actor], [], relu=False)

    return o_m, o_d, o_a


# speedup vs baseline: 4.0196x; 1.1941x over previous
import functools

import jax
import jax.numpy as jnp
from jax import lax
from jax.experimental import pallas as pl
from jax.experimental.pallas import tpu as pltpu
from jax.experimental.pallas import tpu_sc as plsc

N_MOVIE, N_DIRECTOR, N_ACTOR = 50000, 10000, 50000
D_IN, H, OUT = 128, 64, 64
HALF = 32

NUM_TILES = 16
LANE = 128
EDGE_ALIGN = NUM_TILES * LANE

ACC_M = 50048
ACC_D = 10240

E_MD = 200704
E_DM = 200704
E_MA = 102400
E_AM = 102400


def _round_up(x, m):
    return (x + m - 1) // m * m


_BN = 1024


def _full(shape):
    return pl.BlockSpec(shape, lambda i: (0,) * len(shape))


def _rows(shape):
    return pl.BlockSpec(shape, lambda i: (i,) + (0,) * (len(shape) - 1))


def _tc_project(x, Ws):
    n = x.shape[0]
    nb = pl.cdiv(n, _BN)

    def body(*refs):
        x_ref = refs[0]
        w_refs = refs[1:-1]
        o_ref = refs[-1]
        xb = x_ref[...]
        zs = [jnp.dot(xb, w_ref[...], preferred_element_type=jnp.float32)
              for w_ref in w_refs]
        if len(zs) == 1:
            o_ref[:, :H] = zs[0]
        else:
            o_ref[...] = jnp.concatenate(zs, axis=1)

    out = pl.pallas_call(
        body,
        grid=(nb,),
        in_specs=[_rows((_BN, x.shape[1]))] + [_full(w.shape) for w in Ws],
        out_specs=_rows((_BN, 128)),
        out_shape=jax.ShapeDtypeStruct((n, 128), jnp.float32),
    )(x, *Ws)
    return out.reshape(4 * n, HALF)


def _tc_combine(agg_lo, agg_hi, x, Ws_list, b_list, W2_list, relu):
    n = x.shape[0]
    nb = pl.cdiv(n, _BN)
    nws, nb_, nw2 = len(Ws_list), len(b_list), len(W2_list)

    def body(*refs):
        agg_lo_ref, agg_hi_ref, x_ref = refs[:3]
        ws_refs = refs[3:3 + nws]
        b_refs = refs[3 + nws:3 + nws + nb_]
        w2_refs = refs[3 + nws + nb_:3 + nws + nb_ + nw2]
        y_ref = refs[3 + nws + nb_ + nw2]
        ws = ws_refs[0][...]
        for r in ws_refs[1:]:
            ws = ws + r[...]
        bb = b_refs[0][...]
        for r in b_refs[1:]:
            bb = bb + r[...]
        agg = jnp.concatenate([agg_lo_ref[...], agg_hi_ref[...]], axis=1)
        y = agg + bb + jnp.dot(x_ref[...], ws, preferred_element_type=jnp.float32)
        if relu:
            y = jnp.maximum(y, 0.0)
        y_ref[...] = y
        if nw2:
            z_ref = refs[3 + nws + nb_ + nw2 + 1]
            zs = [jnp.dot(y, w2_ref[...], preferred_element_type=jnp.float32)
                  for w2_ref in w2_refs]
            if len(zs) == 1:
                z_ref[:, :H] = zs[0]
            else:
                z_ref[...] = jnp.concatenate(zs, axis=1)

    in_specs = ([_rows((_BN, HALF))] * 2 + [_rows((_BN, x.shape[1]))]
                + [_full(w.shape) for w in Ws_list]
                + [_full((1, H))] * nb_
                + [_full(w.shape) for w in W2_list])
    out_shape = [jax.ShapeDtypeStruct((n, H), jnp.float32)]
    out_specs = [_rows((_BN, H))]
    if nw2:
        out_shape.append(jax.ShapeDtypeStruct((n, 128), jnp.float32))
        out_specs.append(_rows((_BN, 128)))
    outs = pl.pallas_call(
        body,
        grid=(nb,),
        in_specs=in_specs,
        out_specs=out_specs,
        out_shape=out_shape,
    )(agg_lo, agg_hi, x, *Ws_list, *[b.reshape(1, H) for b in b_list], *W2_list)
    y = outs[0]
    z = outs[1].reshape(4 * n, HALF) if nw2 else None
    return y, z


def _sc_aggregate(edges, zv_movie, zv_director, zv_actor):
    mesh = plsc.VectorSubcoreMesh(core_axis_name="c", subcore_axis_name="s")

    rpt_md = E_MD // EDGE_ALIGN
    rpt_dm = E_DM // EDGE_ALIGN
    rpt_ma = E_MA // EDGE_ALIGN
    rpt_am = E_AM // EDGE_ALIGN

    def body(ei_md0, ei_md1, ei_dm0, ei_dm1, ei_ma0, ei_ma1, ei_am0, ei_am1,
             zv_m, zv_d, zv_a,
             out_m_lo, out_m_hi, out_d_lo, out_d_hi, out_a_lo, out_a_hi,
             acc, zbuf, ibuf, rowbuf, isem, gsem):
        cid = lax.axis_index("c")
        sid = lax.axis_index("s")
        zero16 = jnp.zeros((16,), jnp.float32)

        def zero_zbuf():
            def zrow(i, carry):
                zbuf[i, pl.ds(0, 16)] = zero16
                zbuf[i, pl.ds(16, 16)] = zero16
                return carry

            lax.fori_loop(0, LANE, zrow, 0)

        def zero_acc(rows_per_tile):
            base = sid * rows_per_tile
            nfull, rem = divmod(rows_per_tile, LANE)
            for k in range(nfull):
                pltpu.sync_copy(zbuf, acc.at[pl.ds(base + k * LANE, LANE)])
            if rem:
                pltpu.sync_copy(zbuf.at[pl.ds(0, rem)],
                                acc.at[pl.ds(base + nfull * LANE, rem)])

        def run_pass(ei_r, z_r, n):
            base = sid * n
            pltpu.async_copy(ei_r.at[base], ibuf.at[0], isem)
            pltpu.async_copy(ei_r.at[base + 1], ibuf.at[1], isem)

            def step(i, carry):
                m = lax.rem(i, 4)
                g = lax.rem(i, 2)
                pltpu.make_async_copy(ei_r.at[base], ibuf.at[0], isem).wait()
                pltpu.async_copy(z_r.at[ibuf.at[m, 0]], rowbuf.at[g], gsem)

                @pl.when(i + 2 < n)
                def _():
                    pltpu.async_copy(ei_r.at[base + i + 2],
                                     ibuf.at[lax.rem(i + 2, 4)], isem)

                @pl.when(i >= 1)
                def _():
                    mp = lax.rem(i + 3, 4)
                    gp = lax.rem(i + 1, 2)
                    pltpu.make_async_copy(z_r.at[pl.ds(0, LANE)],
                                          rowbuf.at[0], gsem).wait()
                    pltpu.sync_copy(rowbuf.at[gp], acc.at[ibuf.at[mp, 1]],
                                    add=True)

                return carry

            lax.fori_loop(0, n, step, 0)
            pltpu.make_async_copy(z_r.at[pl.ds(0, LANE)],
                                  rowbuf.at[0], gsem).wait()
            pltpu.sync_copy(rowbuf.at[(n - 1) % 2],
                            acc.at[ibuf.at[(n - 1) % 4, 1]], add=True)

        def drain(out_r, rows_per_tile):
            base = sid * rows_per_tile
            pltpu.sync_copy(acc.at[pl.ds(base, rows_per_tile)],
                            out_r.at[pl.ds(base, rows_per_tile)])

        def run_all(ei_dm, ei_am, ei_md, ei_ma, out_m, out_d, out_a):
            zero_zbuf()
            zero_acc(ACC_M // NUM_TILES)
            plsc.subcore_barrier()
            run_pass(ei_dm, zv_d, rpt_dm)
            run_pass(ei_am, zv_a, rpt_am)
            plsc.subcore_barrier()
            drain(out_m, ACC_M // NUM_TILES)
            plsc.subcore_barrier()
            zero_acc(ACC_D // NUM_TILES)
            plsc.subcore_barrier()
            run_pass(ei_md, zv_m, rpt_md)
            plsc.subcore_barrier()
            drain(out_d, ACC_D // NUM_TILES)
            plsc.subcore_barrier()
            zero_acc(ACC_M // NUM_TILES)
            plsc.subcore_barrier()
            run_pass(ei_ma, zv_m, rpt_ma)
            plsc.subcore_barrier()
            drain(out_a, ACC_M // NUM_TILES)

        @pl.when(cid == 0)
        def _():
            run_all(ei_dm0, ei_am0, ei_md0, ei_ma0, out_m_lo, out_d_lo, out_a_lo)

        @pl.when(cid == 1)
        def _():
            run_all(ei_dm1, ei_am1, ei_md1, ei_ma1, out_m_hi, out_d_hi, out_a_hi)

    out_type = [
        jax.ShapeDtypeStruct((ACC_M, HALF), jnp.float32),
        jax.ShapeDtypeStruct((ACC_M, HALF), jnp.float32),
        jax.ShapeDtypeStruct((ACC_D, HALF), jnp.float32),
        jax.ShapeDtypeStruct((ACC_D, HALF), jnp.float32),
        jax.ShapeDtypeStruct((ACC_M, HALF), jnp.float32),
        jax.ShapeDtypeStruct((ACC_M, HALF), jnp.float32),
    ]
    scratch_types = [
        pltpu.VMEM_SHARED((ACC_M, HALF), jnp.float32),
        pltpu.VMEM((LANE, HALF), jnp.float32),
        pltpu.VMEM((4, 2, LANE), jnp.int32),
        pltpu.VMEM((2, LANE, HALF), jnp.float32),
        pltpu.SemaphoreType.DMA,
        pltpu.SemaphoreType.DMA,
    ]
    fn = pl.kernel(body, out_type=out_type, mesh=mesh,
                   scratch_types=scratch_types,
                   compiler_params=pltpu.CompilerParams(
                       use_tc_tiling_on_sc=False))
    outs = fn(*edges[("movie", "director")], *edges[("director", "movie")],
              *edges[("movie", "actor")], *edges[("actor", "movie")],
              zv_movie, zv_director, zv_actor)
    return (outs[0], outs[1]), (outs[2], outs[3]), (outs[4], outs[5])


def _prep_edges(ei, e_pad, dummy, slot):
    e = ei.shape[1]
    src4 = 4 * jnp.pad(ei[0], (0, e_pad - e)) + slot
    dst = jnp.pad(ei[1], (0, e_pad - e), constant_values=dummy)
    dst2 = dst.reshape(-1, LANE)
    return (jnp.stack([src4.reshape(-1, LANE), dst2], axis=1),
            jnp.stack([(src4 + 1).reshape(-1, LANE), dst2], axis=1))


def kernel(x_movie, x_director, x_actor, ei_movie_director, ei_director_movie, ei_movie_actor, ei_actor_movie, W1r_movie_director, b1_movie_director, W1s_movie_director, W2r_movie_director, b2_movie_director, W2s_movie_director, W1r_director_movie, b1_director_movie, W1s_director_movie, W2r_director_movie, b2_director_movie, W2s_director_movie, W1r_movie_actor, b1_movie_actor, W1s_movie_actor, W2r_movie_actor, b2_movie_actor, W2s_movie_actor, W1r_actor_movie, b1_actor_movie, W1s_actor_movie, W2r_actor_movie, b2_actor_movie, W2s_actor_movie):
    edges = {
        ("movie", "director"): _prep_edges(ei_movie_director, E_MD, N_DIRECTOR, 0),
        ("director", "movie"): _prep_edges(ei_director_movie, E_DM, N_MOVIE, 0),
        ("movie", "actor"): _prep_edges(ei_movie_actor, E_MA, N_ACTOR, 2),
        ("actor", "movie"): _prep_edges(ei_actor_movie, E_AM, N_MOVIE, 0),
    }

    zv1_m = _tc_project(x_movie, [W1r_movie_director, W1r_movie_actor])
    zv1_d = _tc_project(x_director, [W1r_director_movie])
    zv1_a = _tc_project(x_actor, [W1r_actor_movie])

    agg1_m, agg1_d, agg1_a = _sc_aggregate(edges, zv1_m, zv1_d, zv1_a)

    h_m, zv2_m = _tc_combine(
        agg1_m[0], agg1_m[1], x_movie,
        [W1s_director_movie, W1s_actor_movie],
        [b1_director_movie, b1_actor_movie],
        [W2r_movie_director, W2r_movie_actor], relu=True)
    h_d, zv2_d = _tc_combine(
        agg1_d[0], agg1_d[1], x_director,
        [W1s_movie_director], [b1_movie_director],
        [W2r_director_movie], relu=True)
    h_a, zv2_a = _tc_combine(
        agg1_a[0], agg1_a[1], x_actor,
        [W1s_movie_actor], [b1_movie_actor],
        [W2r_actor_movie], relu=True)

    agg2_m, agg2_d, agg2_a = _sc_aggregate(edges, zv2_m, zv2_d, zv2_a)

    o_m, _ = _tc_combine(
        agg2_m[0], agg2_m[1], h_m,
        [W2s_director_movie, W2s_actor_movie],
        [b2_director_movie, b2_actor_movie], [], relu=False)
    o_d, _ = _tc_combine(
        agg2_d[0], agg2_d[1], h_d,
        [W2s_movie_director], [b2_movie_director], [], relu=False)
    o_a, _ = _tc_combine(
        agg2_a[0], agg2_a[1], h_a,
        [W2s_movie_actor], [b2_movie_actor], [], relu=False)

    return o_m, o_d, o_a


# speedup vs baseline: 4.1732x; 1.0382x over previous
import functools

import jax
import jax.numpy as jnp
from jax import lax
from jax.experimental import pallas as pl
from jax.experimental.pallas import tpu as pltpu
from jax.experimental.pallas import tpu_sc as plsc

N_MOVIE, N_DIRECTOR, N_ACTOR = 50000, 10000, 50000
D_IN, H, OUT = 128, 64, 64
HALF = 32

NUM_TILES = 16
LANE = 128
EDGE_ALIGN = NUM_TILES * LANE

ACC_M = 50048
ACC_D = 10240

E_MD = 200704
E_DM = 200704
E_MA = 102400
E_AM = 102400


def _round_up(x, m):
    return (x + m - 1) // m * m


_BN = 1024


def _full(shape):
    return pl.BlockSpec(shape, lambda i: (0,) * len(shape))


def _rows(shape):
    return pl.BlockSpec(shape, lambda i: (i,) + (0,) * (len(shape) - 1))


def _tc_project(x, Ws):
    n = x.shape[0]
    nb = pl.cdiv(n, _BN)

    def body(*refs):
        x_ref = refs[0]
        w_refs = refs[1:-1]
        o_ref = refs[-1]
        xb = x_ref[...]
        zs = [jnp.dot(xb, w_ref[...], preferred_element_type=jnp.float32)
              for w_ref in w_refs]
        if len(zs) == 1:
            o_ref[:, :H] = zs[0]
        else:
            o_ref[...] = jnp.concatenate(zs, axis=1)

    out = pl.pallas_call(
        body,
        grid=(nb,),
        in_specs=[_rows((_BN, x.shape[1]))] + [_full(w.shape) for w in Ws],
        out_specs=_rows((_BN, 128)),
        out_shape=jax.ShapeDtypeStruct((n, 128), jnp.float32),
    )(x, *Ws)
    return out.reshape(4 * n, HALF)


def _tc_combine(agg_lo, agg_hi, x, Ws_list, b_list, W2_list, relu):
    n = x.shape[0]
    nb = pl.cdiv(n, _BN)
    nws, nb_, nw2 = len(Ws_list), len(b_list), len(W2_list)

    def body(*refs):
        agg_lo_ref, agg_hi_ref, x_ref = refs[:3]
        ws_refs = refs[3:3 + nws]
        b_refs = refs[3 + nws:3 + nws + nb_]
        w2_refs = refs[3 + nws + nb_:3 + nws + nb_ + nw2]
        y_ref = refs[3 + nws + nb_ + nw2]
        ws = ws_refs[0][...]
        for r in ws_refs[1:]:
            ws = ws + r[...]
        bb = b_refs[0][...]
        for r in b_refs[1:]:
            bb = bb + r[...]
        agg = jnp.concatenate([agg_lo_ref[...], agg_hi_ref[...]], axis=1)
        y = agg + bb + jnp.dot(x_ref[...], ws, preferred_element_type=jnp.float32)
        if relu:
            y = jnp.maximum(y, 0.0)
        y_ref[...] = y
        if nw2:
            z_ref = refs[3 + nws + nb_ + nw2 + 1]
            zs = [jnp.dot(y, w2_ref[...], preferred_element_type=jnp.float32)
                  for w2_ref in w2_refs]
            if len(zs) == 1:
                z_ref[:, :H] = zs[0]
            else:
                z_ref[...] = jnp.concatenate(zs, axis=1)

    in_specs = ([_rows((_BN, HALF))] * 2 + [_rows((_BN, x.shape[1]))]
                + [_full(w.shape) for w in Ws_list]
                + [_full((1, H))] * nb_
                + [_full(w.shape) for w in W2_list])
    out_shape = [jax.ShapeDtypeStruct((n, H), jnp.float32)]
    out_specs = [_rows((_BN, H))]
    if nw2:
        out_shape.append(jax.ShapeDtypeStruct((n, 128), jnp.float32))
        out_specs.append(_rows((_BN, 128)))
    outs = pl.pallas_call(
        body,
        grid=(nb,),
        in_specs=in_specs,
        out_specs=out_specs,
        out_shape=out_shape,
    )(agg_lo, agg_hi, x, *Ws_list, *[b.reshape(1, H) for b in b_list], *W2_list)
    y = outs[0]
    z = outs[1].reshape(4 * n, HALF) if nw2 else None
    return y, z


def _sc_aggregate(edges, zv_movie, zv_director, zv_actor):
    mesh = plsc.VectorSubcoreMesh(core_axis_name="c", subcore_axis_name="s")

    rpt_md = E_MD // EDGE_ALIGN
    rpt_dm = E_DM // EDGE_ALIGN
    rpt_ma = E_MA // EDGE_ALIGN
    rpt_am = E_AM // EDGE_ALIGN

    def body(ei_md0, ei_md1, ei_dm0, ei_dm1, ei_ma0, ei_ma1, ei_am0, ei_am1,
             zv_m, zv_d, zv_a,
             out_m_lo, out_m_hi, out_d_lo, out_d_hi, out_a_lo, out_a_hi,
             acc, zbuf, ibuf, rowbuf, isem, gsem, ssem):
        cid = lax.axis_index("c")
        sid = lax.axis_index("s")
        zero16 = jnp.zeros((16,), jnp.float32)

        def zero_zbuf():
            def zrow(i, carry):
                zbuf[i, pl.ds(0, 16)] = zero16
                zbuf[i, pl.ds(16, 16)] = zero16
                return carry

            lax.fori_loop(0, LANE, zrow, 0)

        def zero_acc(rows_per_tile):
            base = sid * rows_per_tile
            nfull, rem = divmod(rows_per_tile, LANE)
            for k in range(nfull):
                pltpu.sync_copy(zbuf, acc.at[pl.ds(base + k * LANE, LANE)])
            if rem:
                pltpu.sync_copy(zbuf.at[pl.ds(0, rem)],
                                acc.at[pl.ds(base + nfull * LANE, rem)])

        def run_pass(ei_r, z_r, n):
            base = sid * n
            pltpu.async_copy(ei_r.at[base], ibuf.at[0], isem)
            pltpu.async_copy(ei_r.at[base + 1], ibuf.at[1], isem)

            def wait_gather():
                pltpu.make_async_copy(z_r.at[pl.ds(0, LANE)],
                                      rowbuf.at[0], gsem).wait()

            def wait_scatter():
                pltpu.make_async_copy(rowbuf.at[0],
                                      acc.at[pl.ds(0, LANE)], ssem).wait()

            def step(i, carry):
                m = lax.rem(i, 4)
                pltpu.make_async_copy(ei_r.at[base], ibuf.at[0], isem).wait()
                pltpu.async_copy(z_r.at[ibuf.at[m, 0]], rowbuf.at[m], gsem)

                @pl.when(i >= 2)
                def _():
                    wait_scatter()

                @pl.when(i + 2 < n)
                def _():
                    pltpu.async_copy(ei_r.at[base + i + 2],
                                     ibuf.at[lax.rem(i + 2, 4)], isem)

                @pl.when(i >= 1)
                def _():
                    mp = lax.rem(i + 3, 4)
                    wait_gather()
                    pltpu.async_copy(rowbuf.at[mp], acc.at[ibuf.at[mp, 1]],
                                     ssem, add=True)

                return carry

            lax.fori_loop(0, n, step, 0)
            wait_gather()
            ml = (n - 1) % 4
            pltpu.async_copy(rowbuf.at[ml], acc.at[ibuf.at[ml, 1]],
                             ssem, add=True)
            wait_scatter()
            wait_scatter()

        def drain(out_r, rows_per_tile):
            base = sid * rows_per_tile
            pltpu.sync_copy(acc.at[pl.ds(base, rows_per_tile)],
                            out_r.at[pl.ds(base, rows_per_tile)])

        def run_all(ei_dm, ei_am, ei_md, ei_ma, out_m, out_d, out_a):
            zero_zbuf()
            zero_acc(ACC_M // NUM_TILES)
            plsc.subcore_barrier()
            run_pass(ei_dm, zv_d, rpt_dm)
            run_pass(ei_am, zv_a, rpt_am)
            plsc.subcore_barrier()
            drain(out_m, ACC_M // NUM_TILES)
            plsc.subcore_barrier()
            zero_acc(ACC_D // NUM_TILES)
            plsc.subcore_barrier()
            run_pass(ei_md, zv_m, rpt_md)
            plsc.subcore_barrier()
            drain(out_d, ACC_D // NUM_TILES)
            plsc.subcore_barrier()
            zero_acc(ACC_M // NUM_TILES)
            plsc.subcore_barrier()
            run_pass(ei_ma, zv_m, rpt_ma)
            plsc.subcore_barrier()
            drain(out_a, ACC_M // NUM_TILES)

        @pl.when(cid == 0)
        def _():
            run_all(ei_dm0, ei_am0, ei_md0, ei_ma0, out_m_lo, out_d_lo, out_a_lo)

        @pl.when(cid == 1)
        def _():
            run_all(ei_dm1, ei_am1, ei_md1, ei_ma1, out_m_hi, out_d_hi, out_a_hi)

    out_type = [
        jax.ShapeDtypeStruct((ACC_M, HALF), jnp.float32),
        jax.ShapeDtypeStruct((ACC_M, HALF), jnp.float32),
        jax.ShapeDtypeStruct((ACC_D, HALF), jnp.float32),
        jax.ShapeDtypeStruct((ACC_D, HALF), jnp.float32),
        jax.ShapeDtypeStruct((ACC_M, HALF), jnp.float32),
        jax.ShapeDtypeStruct((ACC_M, HALF), jnp.float32),
    ]
    scratch_types = [
        pltpu.VMEM_SHARED((ACC_M, HALF), jnp.float32),
        pltpu.VMEM((LANE, HALF), jnp.float32),
        pltpu.VMEM((4, 2, LANE), jnp.int32),
        pltpu.VMEM((4, LANE, HALF), jnp.float32),
        pltpu.SemaphoreType.DMA,
        pltpu.SemaphoreType.DMA,
        pltpu.SemaphoreType.DMA,
    ]
    fn = pl.kernel(body, out_type=out_type, mesh=mesh,
                   scratch_types=scratch_types,
                   compiler_params=pltpu.CompilerParams(
                       use_tc_tiling_on_sc=False))
    outs = fn(*edges[("movie", "director")], *edges[("director", "movie")],
              *edges[("movie", "actor")], *edges[("actor", "movie")],
              zv_movie, zv_director, zv_actor)
    return (outs[0], outs[1]), (outs[2], outs[3]), (outs[4], outs[5])


def _prep_edges(ei, e_pad, dummy, slot):
    e = ei.shape[1]
    src4 = 4 * jnp.pad(ei[0], (0, e_pad - e)) + slot
    dst = jnp.pad(ei[1], (0, e_pad - e), constant_values=dummy)
    dst2 = dst.reshape(-1, LANE)
    return (jnp.stack([src4.reshape(-1, LANE), dst2], axis=1),
            jnp.stack([(src4 + 1).reshape(-1, LANE), dst2], axis=1))


def kernel(x_movie, x_director, x_actor, ei_movie_director, ei_director_movie, ei_movie_actor, ei_actor_movie, W1r_movie_director, b1_movie_director, W1s_movie_director, W2r_movie_director, b2_movie_director, W2s_movie_director, W1r_director_movie, b1_director_movie, W1s_director_movie, W2r_director_movie, b2_director_movie, W2s_director_movie, W1r_movie_actor, b1_movie_actor, W1s_movie_actor, W2r_movie_actor, b2_movie_actor, W2s_movie_actor, W1r_actor_movie, b1_actor_movie, W1s_actor_movie, W2r_actor_movie, b2_actor_movie, W2s_actor_movie):
    edges = {
        ("movie", "director"): _prep_edges(ei_movie_director, E_MD, N_DIRECTOR, 0),
        ("director", "movie"): _prep_edges(ei_director_movie, E_DM, N_MOVIE, 0),
        ("movie", "actor"): _prep_edges(ei_movie_actor, E_MA, N_ACTOR, 2),
        ("actor", "movie"): _prep_edges(ei_actor_movie, E_AM, N_MOVIE, 0),
    }

    zv1_m = _tc_project(x_movie, [W1r_movie_director, W1r_movie_actor])
    zv1_d = _tc_project(x_director, [W1r_director_movie])
    zv1_a = _tc_project(x_actor, [W1r_actor_movie])

    agg1_m, agg1_d, agg1_a = _sc_aggregate(edges, zv1_m, zv1_d, zv1_a)

    h_m, zv2_m = _tc_combine(
        agg1_m[0], agg1_m[1], x_movie,
        [W1s_director_movie, W1s_actor_movie],
        [b1_director_movie, b1_actor_movie],
        [W2r_movie_director, W2r_movie_actor], relu=True)
    h_d, zv2_d = _tc_combine(
        agg1_d[0], agg1_d[1], x_director,
        [W1s_movie_director], [b1_movie_director],
        [W2r_director_movie], relu=True)
    h_a, zv2_a = _tc_combine(
        agg1_a[0], agg1_a[1], x_actor,
        [W1s_movie_actor], [b1_movie_actor],
        [W2r_actor_movie], relu=True)

    agg2_m, agg2_d, agg2_a = _sc_aggregate(edges, zv2_m, zv2_d, zv2_a)

    o_m, _ = _tc_combine(
        agg2_m[0], agg2_m[1], h_m,
        [W2s_director_movie, W2s_actor_movie],
        [b2_director_movie, b2_actor_movie], [], relu=False)
    o_d, _ = _tc_combine(
        agg2_d[0], agg2_d[1], h_d,
        [W2s_movie_director], [b2_movie_director], [], relu=False)
    o_a, _ = _tc_combine(
        agg2_a[0], agg2_a[1], h_a,
        [W2s_movie_actor], [b2_movie_actor], [], relu=False)

    return o_m, o_d, o_a


# speedup vs baseline: 4.9914x; 1.1961x over previous
import functools

import jax
import jax.numpy as jnp
from jax import lax
from jax.experimental import pallas as pl
from jax.experimental.pallas import tpu as pltpu
from jax.experimental.pallas import tpu_sc as plsc

N_MOVIE, N_DIRECTOR, N_ACTOR = 50000, 10000, 50000
D_IN, H, OUT = 128, 64, 64
HALF = 32

NUM_TILES = 16
LANE = 128
EDGE_ALIGN = NUM_TILES * LANE

ACC_M = 50048
ACC_D = 10240

E_MD = 200704
E_DM = 200704
E_MA = 102400
E_AM = 102400


def _round_up(x, m):
    return (x + m - 1) // m * m


_BN = 1024


def _full(shape):
    return pl.BlockSpec(shape, lambda i: (0,) * len(shape))


def _rows(shape):
    return pl.BlockSpec(shape, lambda i: (i,) + (0,) * (len(shape) - 1))


def _tc_project(x, Ws):
    n = x.shape[0]
    nb = pl.cdiv(n, _BN)

    def body(*refs):
        x_ref = refs[0]
        w_refs = refs[1:-1]
        o_ref = refs[-1]
        xb = x_ref[...]
        zs = [jnp.dot(xb, w_ref[...], preferred_element_type=jnp.float32)
              for w_ref in w_refs]
        if len(zs) == 1:
            o_ref[:, :H] = zs[0]
        else:
            o_ref[...] = jnp.concatenate(zs, axis=1)

    out = pl.pallas_call(
        body,
        grid=(nb,),
        in_specs=[_rows((_BN, x.shape[1]))] + [_full(w.shape) for w in Ws],
        out_specs=_rows((_BN, 128)),
        out_shape=jax.ShapeDtypeStruct((n, 128), jnp.float32),
    )(x, *Ws)
    return out.reshape(4 * n, HALF)


def _tc_combine(agg_pk, x, Ws_list, b_list, W2_list, relu, x_transposed):
    n = x.shape[1] if x_transposed else x.shape[0]
    nb = pl.cdiv(n, _BN)
    nws, nb_, nw2 = len(Ws_list), len(b_list), len(W2_list)

    def body(*refs):
        agg_ref, x_ref = refs[:2]
        ws_refs = refs[2:2 + nws]
        b_refs = refs[2 + nws:2 + nws + nb_]
        w2_refs = refs[2 + nws + nb_:2 + nws + nb_ + nw2]
        y_ref = refs[2 + nws + nb_ + nw2]
        ws = ws_refs[0][...]
        for r in ws_refs[1:]:
            ws = ws + r[...]
        bb = b_refs[0][...]
        for r in b_refs[1:]:
            bb = bb + r[...]
        if x_transposed:
            xw = jax.lax.dot_general(
                x_ref[...], ws, (((0,), (0,)), ((), ())),
                preferred_element_type=jnp.float32)
        else:
            xw = jnp.dot(x_ref[...], ws, preferred_element_type=jnp.float32)
        y = agg_ref[...][:, :H] + bb + xw
        if relu:
            y = jnp.maximum(y, 0.0)
        y_ref[...] = y.T
        if nw2:
            z_ref = refs[2 + nws + nb_ + nw2 + 1]
            zs = [jnp.dot(y, w2_ref[...], preferred_element_type=jnp.float32)
                  for w2_ref in w2_refs]
            if len(zs) == 1:
                z_ref[:, :H] = zs[0]
            else:
                z_ref[...] = jnp.concatenate(zs, axis=1)

    x_spec = (pl.BlockSpec((H, _BN), lambda i: (0, i)) if x_transposed
              else _rows((_BN, x.shape[1])))
    in_specs = ([_rows((_BN, 128)), x_spec]
                + [_full(w.shape) for w in Ws_list]
                + [_full((1, H))] * nb_
                + [_full(w.shape) for w in W2_list])
    out_shape = [jax.ShapeDtypeStruct((H, n), jnp.float32)]
    out_specs = [pl.BlockSpec((H, _BN), lambda i: (0, i))]
    if nw2:
        out_shape.append(jax.ShapeDtypeStruct((n, 128), jnp.float32))
        out_specs.append(_rows((_BN, 128)))
    outs = pl.pallas_call(
        body,
        grid=(nb,),
        in_specs=in_specs,
        out_specs=out_specs,
        out_shape=out_shape,
    )(agg_pk, x, *Ws_list, *[b.reshape(1, H) for b in b_list], *W2_list)
    yT = outs[0]
    z = outs[1].reshape(4 * n, HALF) if nw2 else None
    return yT, z


def _sc_aggregate(edges, zv_movie, zv_director, zv_actor):
    mesh = plsc.VectorSubcoreMesh(core_axis_name="c", subcore_axis_name="s")

    rpt_md = E_MD // EDGE_ALIGN
    rpt_dm = E_DM // EDGE_ALIGN
    rpt_ma = E_MA // EDGE_ALIGN
    rpt_am = E_AM // EDGE_ALIGN

    def body(ei_md0, ei_md1, ei_dm0, ei_dm1, ei_ma0, ei_ma1, ei_am0, ei_am1,
             zv_m, zv_d, zv_a, out_m, out_d, out_a,
             acc, zbuf, ibuf, rowbuf, isem, gsem, ssem):
        cid = lax.axis_index("c")
        sid = lax.axis_index("s")
        zero16 = jnp.zeros((16,), jnp.float32)

        def zero_zbuf():
            def zrow(i, carry):
                zbuf[i, pl.ds(0, 16)] = zero16
                zbuf[i, pl.ds(16, 16)] = zero16
                return carry

            lax.fori_loop(0, LANE, zrow, 0)

        def zero_acc(rows_per_tile):
            base = sid * rows_per_tile
            nfull, rem = divmod(rows_per_tile, LANE)
            for k in range(nfull):
                pltpu.sync_copy(zbuf, acc.at[pl.ds(base + k * LANE, LANE)])
            if rem:
                pltpu.sync_copy(zbuf.at[pl.ds(0, rem)],
                                acc.at[pl.ds(base + nfull * LANE, rem)])

        def run_pass(ei_r, z_r, n):
            base = sid * n
            pltpu.async_copy(ei_r.at[base], ibuf.at[0], isem)
            pltpu.async_copy(ei_r.at[base + 1], ibuf.at[1], isem)

            def wait_gather():
                pltpu.make_async_copy(z_r.at[pl.ds(0, LANE)],
                                      rowbuf.at[0], gsem).wait()

            def wait_scatter():
                pltpu.make_async_copy(rowbuf.at[0],
                                      acc.at[pl.ds(0, LANE)], ssem).wait()

            def step(i, carry):
                m = lax.rem(i, 4)
                pltpu.make_async_copy(ei_r.at[base], ibuf.at[0], isem).wait()
                pltpu.async_copy(z_r.at[ibuf.at[m, 0]], rowbuf.at[m], gsem)

                @pl.when(i >= 2)
                def _():
                    wait_scatter()

                @pl.when(i + 2 < n)
                def _():
                    pltpu.async_copy(ei_r.at[base + i + 2],
                                     ibuf.at[lax.rem(i + 2, 4)], isem)

                @pl.when(i >= 1)
                def _():
                    mp = lax.rem(i + 3, 4)
                    wait_gather()
                    pltpu.async_copy(rowbuf.at[mp], acc.at[ibuf.at[mp, 1]],
                                     ssem, add=True)

                return carry

            lax.fori_loop(0, n, step, 0)
            wait_gather()
            ml = (n - 1) % 4
            pltpu.async_copy(rowbuf.at[ml], acc.at[ibuf.at[ml, 1]],
                             ssem, add=True)
            wait_scatter()
            wait_scatter()

        def drain(out_r, rows_per_tile, c):
            base = sid * rows_per_tile
            pltpu.sync_copy(acc.at[pl.ds(base, rows_per_tile)],
                            out_r.at[pl.ds(base, rows_per_tile),
                                     pl.ds(HALF * c, HALF)])

        def run_all(ei_dm, ei_am, ei_md, ei_ma, c):
            zero_zbuf()
            zero_acc(ACC_M // NUM_TILES)
            plsc.subcore_barrier()
            run_pass(ei_dm, zv_d, rpt_dm)
            run_pass(ei_am, zv_a, rpt_am)
            plsc.subcore_barrier()
            drain(out_m, ACC_M // NUM_TILES, c)
            plsc.subcore_barrier()
            zero_acc(ACC_D // NUM_TILES)
            plsc.subcore_barrier()
            run_pass(ei_md, zv_m, rpt_md)
            plsc.subcore_barrier()
            drain(out_d, ACC_D // NUM_TILES, c)
            plsc.subcore_barrier()
            zero_acc(ACC_M // NUM_TILES)
            plsc.subcore_barrier()
            run_pass(ei_ma, zv_m, rpt_ma)
            plsc.subcore_barrier()
            drain(out_a, ACC_M // NUM_TILES, c)

        @pl.when(cid == 0)
        def _():
            run_all(ei_dm0, ei_am0, ei_md0, ei_ma0, 0)

        @pl.when(cid == 1)
        def _():
            run_all(ei_dm1, ei_am1, ei_md1, ei_ma1, 1)

    out_type = [
        jax.ShapeDtypeStruct((ACC_M, 128), jnp.float32),
        jax.ShapeDtypeStruct((ACC_D, 128), jnp.float32),
        jax.ShapeDtypeStruct((ACC_M, 128), jnp.float32),
    ]
    scratch_types = [
        pltpu.VMEM_SHARED((ACC_M, HALF), jnp.float32),
        pltpu.VMEM((LANE, HALF), jnp.float32),
        pltpu.VMEM((4, 2, LANE), jnp.int32),
        pltpu.VMEM((4, LANE, HALF), jnp.float32),
        pltpu.SemaphoreType.DMA,
        pltpu.SemaphoreType.DMA,
        pltpu.SemaphoreType.DMA,
    ]
    fn = pl.kernel(body, out_type=out_type, mesh=mesh,
                   scratch_types=scratch_types,
                   compiler_params=pltpu.CompilerParams(
                       use_tc_tiling_on_sc=False))
    outs = fn(*edges[("movie", "director")], *edges[("director", "movie")],
              *edges[("movie", "actor")], *edges[("actor", "movie")],
              zv_movie, zv_director, zv_actor)
    return outs[0], outs[1], outs[2]


def _prep_edges(ei, e_pad, dummy, slot):
    e = ei.shape[1]
    src4 = 4 * jnp.pad(ei[0], (0, e_pad - e)) + slot
    dst = jnp.pad(ei[1], (0, e_pad - e), constant_values=dummy)
    dst2 = dst.reshape(-1, LANE)
    return (jnp.stack([src4.reshape(-1, LANE), dst2], axis=1),
            jnp.stack([(src4 + 1).reshape(-1, LANE), dst2], axis=1))


def kernel(x_movie, x_director, x_actor, ei_movie_director, ei_director_movie, ei_movie_actor, ei_actor_movie, W1r_movie_director, b1_movie_director, W1s_movie_director, W2r_movie_director, b2_movie_director, W2s_movie_director, W1r_director_movie, b1_director_movie, W1s_director_movie, W2r_director_movie, b2_director_movie, W2s_director_movie, W1r_movie_actor, b1_movie_actor, W1s_movie_actor, W2r_movie_actor, b2_movie_actor, W2s_movie_actor, W1r_actor_movie, b1_actor_movie, W1s_actor_movie, W2r_actor_movie, b2_actor_movie, W2s_actor_movie):
    edges = {
        ("movie", "director"): _prep_edges(ei_movie_director, E_MD, N_DIRECTOR, 0),
        ("director", "movie"): _prep_edges(ei_director_movie, E_DM, N_MOVIE, 0),
        ("movie", "actor"): _prep_edges(ei_movie_actor, E_MA, N_ACTOR, 2),
        ("actor", "movie"): _prep_edges(ei_actor_movie, E_AM, N_MOVIE, 0),
    }

    zv1_m = _tc_project(x_movie, [W1r_movie_director, W1r_movie_actor])
    zv1_d = _tc_project(x_director, [W1r_director_movie])
    zv1_a = _tc_project(x_actor, [W1r_actor_movie])

    agg1_m, agg1_d, agg1_a = _sc_aggregate(edges, zv1_m, zv1_d, zv1_a)

    hT_m, zv2_m = _tc_combine(
        agg1_m, x_movie,
        [W1s_director_movie, W1s_actor_movie],
        [b1_director_movie, b1_actor_movie],
        [W2r_movie_director, W2r_movie_actor], relu=True, x_transposed=False)
    hT_d, zv2_d = _tc_combine(
        agg1_d, x_director,
        [W1s_movie_director], [b1_movie_director],
        [W2r_director_movie], relu=True, x_transposed=False)
    hT_a, zv2_a = _tc_combine(
        agg1_a, x_actor,
        [W1s_movie_actor], [b1_movie_actor],
        [W2r_actor_movie], relu=True, x_transposed=False)

    agg2_m, agg2_d, agg2_a = _sc_aggregate(edges, zv2_m, zv2_d, zv2_a)

    oT_m, _ = _tc_combine(
        agg2_m, hT_m,
        [W2s_director_movie, W2s_actor_movie],
        [b2_director_movie, b2_actor_movie], [], relu=False, x_transposed=True)
    oT_d, _ = _tc_combine(
        agg2_d, hT_d,
        [W2s_movie_director], [b2_movie_director], [], relu=False,
        x_transposed=True)
    oT_a, _ = _tc_combine(
        agg2_a, hT_a,
        [W2s_movie_actor], [b2_movie_actor], [], relu=False, x_transposed=True)

    return oT_m.T, oT_d.T, oT_a.T


# speedup vs baseline: 6.1560x; 1.2333x over previous
import functools

import jax
import jax.numpy as jnp
from jax import lax
from jax.experimental import pallas as pl
from jax.experimental.pallas import tpu as pltpu
from jax.experimental.pallas import tpu_sc as plsc

N_MOVIE, N_DIRECTOR, N_ACTOR = 50000, 10000, 50000
D_IN, H, OUT = 128, 64, 64
HALF = 32

NUM_TILES = 16
LANE = 128
EDGE_ALIGN = NUM_TILES * LANE

ACC_M = 50048
ACC_D = 10240

E_MD = 200704
E_DM = 200704
E_MA = 102400
E_AM = 102400


def _round_up(x, m):
    return (x + m - 1) // m * m


_BN = 1024


def _full(shape):
    return pl.BlockSpec(shape, lambda i: (0,) * len(shape))


def _rows(shape):
    return pl.BlockSpec(shape, lambda i: (i,) + (0,) * (len(shape) - 1))


def _tc_project(x, Ws):
    n = x.shape[0]
    nb = pl.cdiv(n, _BN)

    def body(*refs):
        x_ref = refs[0]
        w_refs = refs[1:-1]
        o_ref = refs[-1]
        xb = x_ref[...]
        zs = [jnp.dot(xb, w_ref[...], preferred_element_type=jnp.float32)
              for w_ref in w_refs]
        if len(zs) == 1:
            o_ref[:, :H] = zs[0]
        else:
            o_ref[...] = jnp.concatenate(zs, axis=1)

    out = pl.pallas_call(
        body,
        grid=(nb,),
        in_specs=[_rows((_BN, x.shape[1]))] + [_full(w.shape) for w in Ws],
        out_specs=_rows((_BN, 128)),
        out_shape=jax.ShapeDtypeStruct((n, 128), jnp.float32),
    )(x, *Ws)
    return out.reshape(4 * n, HALF)


def _tc_combine(agg_pk, x, Ws_list, b_list, W2_list, relu, x_transposed):
    n = x.shape[1] if x_transposed else x.shape[0]
    nb = pl.cdiv(n, _BN)
    nws, nb_, nw2 = len(Ws_list), len(b_list), len(W2_list)

    def body(*refs):
        agg_ref, x_ref = refs[:2]
        ws_refs = refs[2:2 + nws]
        b_refs = refs[2 + nws:2 + nws + nb_]
        w2_refs = refs[2 + nws + nb_:2 + nws + nb_ + nw2]
        y_ref = refs[2 + nws + nb_ + nw2]
        ws = ws_refs[0][...]
        for r in ws_refs[1:]:
            ws = ws + r[...]
        bb = b_refs[0][...]
        for r in b_refs[1:]:
            bb = bb + r[...]
        if x_transposed:
            xw = jax.lax.dot_general(
                x_ref[...], ws, (((0,), (0,)), ((), ())),
                preferred_element_type=jnp.float32)
        else:
            xw = jnp.dot(x_ref[...], ws, preferred_element_type=jnp.float32)
        y = agg_ref[...][:, :H] + bb + xw
        if relu:
            y = jnp.maximum(y, 0.0)
        y_ref[...] = y.T
        if nw2:
            z_ref = refs[2 + nws + nb_ + nw2 + 1]
            zs = [jnp.dot(y, w2_ref[...], preferred_element_type=jnp.float32)
                  for w2_ref in w2_refs]
            if len(zs) == 1:
                z_ref[:, :H] = zs[0]
            else:
                z_ref[...] = jnp.concatenate(zs, axis=1)

    x_spec = (pl.BlockSpec((H, _BN), lambda i: (0, i)) if x_transposed
              else _rows((_BN, x.shape[1])))
    in_specs = ([_rows((_BN, 128)), x_spec]
                + [_full(w.shape) for w in Ws_list]
                + [_full((1, H))] * nb_
                + [_full(w.shape) for w in W2_list])
    out_shape = [jax.ShapeDtypeStruct((H, n), jnp.float32)]
    out_specs = [pl.BlockSpec((H, _BN), lambda i: (0, i))]
    if nw2:
        out_shape.append(jax.ShapeDtypeStruct((n, 128), jnp.float32))
        out_specs.append(_rows((_BN, 128)))
    outs = pl.pallas_call(
        body,
        grid=(nb,),
        in_specs=in_specs,
        out_specs=out_specs,
        out_shape=out_shape,
    )(agg_pk, x, *Ws_list, *[b.reshape(1, H) for b in b_list], *W2_list)
    yT = outs[0]
    z = outs[1].reshape(4 * n, HALF) if nw2 else None
    return yT, z


def _sc_kernel(body, out_type):
    mesh = plsc.VectorSubcoreMesh(core_axis_name="c", subcore_axis_name="s")
    scratch_types = [
        pltpu.VMEM_SHARED((ACC_M, HALF), jnp.float32),
        pltpu.VMEM((LANE, HALF), jnp.float32),
        pltpu.VMEM((4, 2, LANE), jnp.int32),
        pltpu.VMEM((4, LANE, HALF), jnp.float32),
        pltpu.SemaphoreType.DMA,
        pltpu.SemaphoreType.DMA,
        pltpu.SemaphoreType.DMA,
    ]
    return pl.kernel(body, out_type=out_type, mesh=mesh,
                     scratch_types=scratch_types,
                     compiler_params=pltpu.CompilerParams(
                         use_tc_tiling_on_sc=False))


def _sc_ops(sid, acc, zbuf, ibuf, rowbuf, isem, gsem, ssem):
    zero16 = jnp.zeros((16,), jnp.float32)

    def zero_zbuf():
        def zrow(i, carry):
            zbuf[i, pl.ds(0, 16)] = zero16
            zbuf[i, pl.ds(16, 16)] = zero16
            return carry

        lax.fori_loop(0, LANE, zrow, 0)

    def zero_acc(rows_per_tile):
        base = sid * rows_per_tile
        nfull, rem = divmod(rows_per_tile, LANE)
        for k in range(nfull):
            pltpu.sync_copy(zbuf, acc.at[pl.ds(base + k * LANE, LANE)])
        if rem:
            pltpu.sync_copy(zbuf.at[pl.ds(0, rem)],
                            acc.at[pl.ds(base + nfull * LANE, rem)])

    def run_pass(ei_r, z_r, n):
        base = sid * n
        pltpu.async_copy(ei_r.at[base], ibuf.at[0], isem)
        pltpu.async_copy(ei_r.at[base + 1], ibuf.at[1], isem)

        def wait_gather():
            pltpu.make_async_copy(z_r.at[pl.ds(0, LANE)],
                                  rowbuf.at[0], gsem).wait()

        def wait_scatter():
            pltpu.make_async_copy(rowbuf.at[0],
                                  acc.at[pl.ds(0, LANE)], ssem).wait()

        def step(i, carry):
            m = lax.rem(i, 4)
            pltpu.make_async_copy(ei_r.at[base], ibuf.at[0], isem).wait()
            pltpu.async_copy(z_r.at[ibuf.at[m, 0]], rowbuf.at[m], gsem)

            @pl.when(i >= 2)
            def _():
                wait_scatter()

            @pl.when(i + 2 < n)
            def _():
                pltpu.async_copy(ei_r.at[base + i + 2],
                                 ibuf.at[lax.rem(i + 2, 4)], isem)

            @pl.when(i >= 1)
            def _():
                mp = lax.rem(i + 3, 4)
                wait_gather()
                pltpu.async_copy(rowbuf.at[mp], acc.at[ibuf.at[mp, 1]],
                                 ssem, add=True)

            return carry

        lax.fori_loop(0, n, step, 0)
        wait_gather()
        ml = (n - 1) % 4
        pltpu.async_copy(rowbuf.at[ml], acc.at[ibuf.at[ml, 1]],
                         ssem, add=True)
        wait_scatter()
        wait_scatter()

    def drain(out_r, rows_per_tile, c):
        base = sid * rows_per_tile
        pltpu.sync_copy(acc.at[pl.ds(base, rows_per_tile)],
                        out_r.at[pl.ds(base, rows_per_tile),
                                 pl.ds(HALF * c, HALF)])

    return zero_zbuf, zero_acc, run_pass, drain


def _sc_agg_movie(edges, zv_d, zv_a):
    rpt_dm = E_DM // EDGE_ALIGN
    rpt_am = E_AM // EDGE_ALIGN

    def body(ei_dm0, ei_dm1, ei_am0, ei_am1, zv_d_r, zv_a_r, out_m,
             acc, zbuf, ibuf, rowbuf, isem, gsem, ssem):
        cid = lax.axis_index("c")
        sid = lax.axis_index("s")
        zero_zbuf, zero_acc, run_pass, drain = _sc_ops(
            sid, acc, zbuf, ibuf, rowbuf, isem, gsem, ssem)

        def run(ei_dm, ei_am, c):
            zero_zbuf()
            zero_acc(ACC_M // NUM_TILES)
            plsc.subcore_barrier()
            run_pass(ei_dm, zv_d_r, rpt_dm)
            run_pass(ei_am, zv_a_r, rpt_am)
            plsc.subcore_barrier()
            drain(out_m, ACC_M // NUM_TILES, c)

        @pl.when(cid == 0)
        def _():
            run(ei_dm0, ei_am0, 0)

        @pl.when(cid == 1)
        def _():
            run(ei_dm1, ei_am1, 1)

    fn = _sc_kernel(body, [jax.ShapeDtypeStruct((ACC_M, 128), jnp.float32)])
    (out,) = fn(*edges[("director", "movie")], *edges[("actor", "movie")],
                zv_d, zv_a)
    return out


def _sc_agg_da(edges, zv_m):
    rpt_md = E_MD // EDGE_ALIGN
    rpt_ma = E_MA // EDGE_ALIGN

    def body(ei_md0, ei_md1, ei_ma0, ei_ma1, zv_m_r, out_d, out_a,
             acc, zbuf, ibuf, rowbuf, isem, gsem, ssem):
        cid = lax.axis_index("c")
        sid = lax.axis_index("s")
        zero_zbuf, zero_acc, run_pass, drain = _sc_ops(
            sid, acc, zbuf, ibuf, rowbuf, isem, gsem, ssem)

        def run(ei_md, ei_ma, c):
            zero_zbuf()
            zero_acc(ACC_D // NUM_TILES)
            plsc.subcore_barrier()
            run_pass(ei_md, zv_m_r, rpt_md)
            plsc.subcore_barrier()
            drain(out_d, ACC_D // NUM_TILES, c)
            plsc.subcore_barrier()
            zero_acc(ACC_M // NUM_TILES)
            plsc.subcore_barrier()
            run_pass(ei_ma, zv_m_r, rpt_ma)
            plsc.subcore_barrier()
            drain(out_a, ACC_M // NUM_TILES, c)

        @pl.when(cid == 0)
        def _():
            run(ei_md0, ei_ma0, 0)

        @pl.when(cid == 1)
        def _():
            run(ei_md1, ei_ma1, 1)

    fn = _sc_kernel(body, [jax.ShapeDtypeStruct((ACC_D, 128), jnp.float32),
                           jax.ShapeDtypeStruct((ACC_M, 128), jnp.float32)])
    out_d, out_a = fn(*edges[("movie", "director")], *edges[("movie", "actor")],
                      zv_m)
    return out_d, out_a


def _prep_edges(ei, e_pad, dummy, slot):
    e = ei.shape[1]
    src4 = 4 * jnp.pad(ei[0], (0, e_pad - e)) + slot
    dst = jnp.pad(ei[1], (0, e_pad - e), constant_values=dummy)
    dst2 = dst.reshape(-1, LANE)
    return (jnp.stack([src4.reshape(-1, LANE), dst2], axis=1),
            jnp.stack([(src4 + 1).reshape(-1, LANE), dst2], axis=1))


def kernel(x_movie, x_director, x_actor, ei_movie_director, ei_director_movie, ei_movie_actor, ei_actor_movie, W1r_movie_director, b1_movie_director, W1s_movie_director, W2r_movie_director, b2_movie_director, W2s_movie_director, W1r_director_movie, b1_director_movie, W1s_director_movie, W2r_director_movie, b2_director_movie, W2s_director_movie, W1r_movie_actor, b1_movie_actor, W1s_movie_actor, W2r_movie_actor, b2_movie_actor, W2s_movie_actor, W1r_actor_movie, b1_actor_movie, W1s_actor_movie, W2r_actor_movie, b2_actor_movie, W2s_actor_movie):
    edges = {
        ("movie", "director"): _prep_edges(ei_movie_director, E_MD, N_DIRECTOR, 0),
        ("director", "movie"): _prep_edges(ei_director_movie, E_DM, N_MOVIE, 0),
        ("movie", "actor"): _prep_edges(ei_movie_actor, E_MA, N_ACTOR, 2),
        ("actor", "movie"): _prep_edges(ei_actor_movie, E_AM, N_MOVIE, 0),
    }

    zv1_m = _tc_project(x_movie, [W1r_movie_director, W1r_movie_actor])
    zv1_d = _tc_project(x_director, [W1r_director_movie])
    zv1_a = _tc_project(x_actor, [W1r_actor_movie])

    agg1_m = _sc_agg_movie(edges, zv1_d, zv1_a)
    agg1_d, agg1_a = _sc_agg_da(edges, zv1_m)

    hT_m, zv2_m = _tc_combine(
        agg1_m, x_movie,
        [W1s_director_movie, W1s_actor_movie],
        [b1_director_movie, b1_actor_movie],
        [W2r_movie_director, W2r_movie_actor], relu=True, x_transposed=False)
    agg2_d, agg2_a = _sc_agg_da(edges, zv2_m)
    hT_d, zv2_d = _tc_combine(
        agg1_d, x_director,
        [W1s_movie_director], [b1_movie_director],
        [W2r_director_movie], relu=True, x_transposed=False)
    hT_a, zv2_a = _tc_combine(
        agg1_a, x_actor,
        [W1s_movie_actor], [b1_movie_actor],
        [W2r_actor_movie], relu=True, x_transposed=False)
    agg2_m = _sc_agg_movie(edges, zv2_d, zv2_a)

    oT_d, _ = _tc_combine(
        agg2_d, hT_d,
        [W2s_movie_director], [b2_movie_director], [], relu=False,
        x_transposed=True)
    oT_a, _ = _tc_combine(
        agg2_a, hT_a,
        [W2s_movie_actor], [b2_movie_actor], [], relu=False, x_transposed=True)
    oT_m, _ = _tc_combine(
        agg2_m, hT_m,
        [W2s_director_movie, W2s_actor_movie],
        [b2_director_movie, b2_actor_movie], [], relu=False, x_transposed=True)

    return oT_m.T, oT_d.T, oT_a.T
